# Initial kernel scaffold; baseline (speedup 1.0000x reference)
import jax
import jax.numpy as jnp
from jax.experimental import pallas as pl


def kernel(x, edge_index, batch, W0, b0, g0, be0, W1, b1, Wr1, g1, be1, W2, b2, Wr2, g2, be2, Wm1, bm1, gm1, bem1, Wm2, bm2):
    raise NotImplementedError("write your pallas kernel here")


# speedup vs baseline: 6.7718x; 6.7718x over previous
from functools import partial

import jax
import jax.numpy as jnp
from jax import lax
from jax.experimental import pallas as pl
from jax.experimental.pallas import tpu as pltpu
from jax.experimental.pallas import tpu_sc as plsc

_N = 10000
_E = 320000
_H = 128
_G = 128
_C = 40
_NC = 2
_NS = 16
_NW = _NC * _NS
_BLK = 128
_PB = 80
_CHK = 40
_EPW = _PB * _BLK
_EP = _NW * _EPW
_RPT = 632
_NR = _NS * _RPT
_EPS = 1e-5


def _sc_mesh():
    return plsc.VectorSubcoreMesh(core_axis_name="c", subcore_axis_name="s",
                                  num_cores=_NC, num_subcores=_NS)


def _deg_partials(dstp, ones128, zeros128):
    @partial(
        pl.kernel,
        out_type=jax.ShapeDtypeStruct((2 * _NR, _H), jnp.float32),
        mesh=_sc_mesh(),
        scratch_types=[
            pltpu.VMEM((_PB, _BLK), jnp.int32),
            pltpu.VMEM((_BLK, _H), jnp.float32),
            pltpu.VMEM_SHARED((_NR, _H), jnp.float32),
        ],
    )
    def deg_kernel(dst_hbm, ones_hbm, zer_hbm, out_hbm, dst_v, buf_v, acc_sh):
        c = lax.axis_index("c")
        s = lax.axis_index("s")
        wid = c * _NS + s
        pltpu.sync_copy(zer_hbm, acc_sh.at[pl.ds(s * _RPT, _RPT)])
        pltpu.sync_copy(ones_hbm, buf_v)
        pltpu.sync_copy(dst_hbm.at[wid], dst_v)
        plsc.subcore_barrier()

        def step(j, carry):
            pltpu.sync_copy(buf_v, acc_sh.at[dst_v.at[j]], add=True)
            return carry

        lax.fori_loop(0, _PB, step, 0)
        plsc.subcore_barrier()
        base = c * _NR + s * _RPT
        pltpu.sync_copy(acc_sh.at[pl.ds(s * _RPT, _RPT)],
                        out_hbm.at[pl.ds(base, _RPT)])

    return deg_kernel(dstp, ones128, zeros128)


def _conv_partials(ap, srcp, dstp, zeros128):
    @partial(
        pl.kernel,
        out_type=jax.ShapeDtypeStruct((2 * _NR, _H), jnp.float32),
        mesh=_sc_mesh(),
        scratch_types=[
            pltpu.VMEM((_CHK, _BLK), jnp.int32),
            pltpu.VMEM((_CHK, _BLK), jnp.int32),
            pltpu.VMEM((_BLK, _H), jnp.float32),
            pltpu.VMEM((_BLK, _H), jnp.float32),
            pltpu.VMEM_SHARED((_NR, _H), jnp.float32),
            pltpu.SemaphoreType.DMA,
            pltpu.SemaphoreType.DMA,
        ],
    )
    def conv_kernel(ap_hbm, src_hbm, dst_hbm, zer_hbm, out_hbm,
                    src_v, dst_v, buf0, buf1, acc_sh, sem0, sem1):
        c = lax.axis_index("c")
        s = lax.axis_index("s")
        wid = c * _NS + s
        pltpu.sync_copy(zer_hbm, acc_sh.at[pl.ds(s * _RPT, _RPT)])
        plsc.subcore_barrier()

        for ch in range(_PB // _CHK):
            pltpu.sync_copy(src_hbm.at[wid, pl.ds(ch * _CHK, _CHK)], src_v)
            pltpu.sync_copy(dst_hbm.at[wid, pl.ds(ch * _CHK, _CHK)], dst_v)

            def step(j, carry):
                pltpu.sync_copy(ap_hbm.at[src_v.at[j]], buf0)
                pltpu.sync_copy(buf0, acc_sh.at[dst_v.at[j]], add=True)
                return carry

            lax.fori_loop(0, _CHK, step, 0)

        plsc.subcore_barrier()
        base = c * _NR + s * _RPT
        pltpu.sync_copy(acc_sh.at[pl.ds(s * _RPT, _RPT)],
                        out_hbm.at[pl.ds(base, _RPT)])

    return conv_kernel(ap, srcp, dstp, zeros128)


def _pre_kernel(deg16, x, W0):
    def body(degr, xr, wr, disr, apr):
        d = degr[...]
        deg = d[0:_N, 0:1] + d[_NR:_NR + _N, 0:1] + 1.0
        dis = lax.rsqrt(deg)
        disr[...] = dis
        a = jnp.dot(xr[...], wr[...], preferred_element_type=jnp.float32)
        apr[...] = a * dis

    return pl.pallas_call(
        body,
        out_shape=[
            jax.ShapeDtypeStruct((_N, 1), jnp.float32),
            jax.ShapeDtypeStruct((_N, _H), jnp.float32),
        ],
    )(deg16, x, W0)


def _mid_kernel(P, ap_prev, carry, dis, g, be, W, Wr, b_next):
    def body(Pr, apr, cr, disr, gr, ber, Wrf, Wrr, bnr, ap_out, carry_out):
        dis = disr[...]
        t = dis * (Pr[0:_N] + Pr[_NR:_NR + _N] + apr[...]) + cr[...]
        m = jnp.mean(t, axis=0)
        v = jnp.mean((t - m) ** 2, axis=0)
        h = jax.nn.relu(gr[...] * (t - m) / jnp.sqrt(v + _EPS) + ber[...])
        ap_out[...] = dis * jnp.dot(h, Wrf[...],
                                    preferred_element_type=jnp.float32)
        carry_out[...] = jnp.dot(h, Wrr[...],
                                 preferred_element_type=jnp.float32) + bnr[...]

    return pl.pallas_call(
        body,
        out_shape=[
            jax.ShapeDtypeStruct((_N, _H), jnp.float32),
            jax.ShapeDtypeStruct((_N, _H), jnp.float32),
        ],
    )(P, ap_prev, carry, dis, g, be, W, Wr, b_next)


def _post_kernel(P, ap2, carry2, dis, g2, be2, batch2d, Wm1, bm1, gm1, bem1, Wm2, bm2):
    def body(Pr, apr, cr, disr, gr, ber, br, Wm1r, bm1r, gm1r, bem1r,
             Wm2r, bm2r, outr, p_ref):
        dis = disr[...]
        t = dis * (Pr[0:_N] + Pr[_NR:_NR + _N] + apr[...]) + cr[...]
        m = jnp.mean(t, axis=0)
        v = jnp.mean((t - m) ** 2, axis=0)
        h = jax.nn.relu(gr[...] * (t - m) / jnp.sqrt(v + _EPS) + ber[...])
        bids = br[...]

        def pool_body(gi, carry):
            mask = bids == gi
            mx = jnp.max(jnp.where(mask, h, -jnp.inf), axis=0)
            p_ref[pl.ds(gi, 1), :] = mx[None, :]
            return carry

        lax.fori_loop(0, _G, pool_body, 0)
        p = p_ref[...]
        m1 = jnp.dot(p, Wm1r[...], preferred_element_type=jnp.float32) + bm1r[...]
        mm = jnp.mean(m1, axis=0)
        vv = jnp.mean((m1 - mm) ** 2, axis=0)
        hm = jax.nn.relu(gm1r[...] * (m1 - mm) / jnp.sqrt(vv + _EPS) + bem1r[...])
        outr[...] = jnp.dot(hm, Wm2r[...],
                            preferred_element_type=jnp.float32) + bm2r[...]

    return pl.pallas_call(
        body,
        out_shape=jax.ShapeDtypeStruct((_G, _C), jnp.float32),
        scratch_shapes=[pltpu.VMEM((_G, _H), jnp.float32)],
    )(P, ap2, carry2, dis, g2, be2, batch2d, Wm1, bm1, gm1, bem1, Wm2, bm2)


def kernel(x, edge_index, batch, W0, b0, g0, be0, W1, b1, Wr1, g1, be1,
           W2, b2, Wr2, g2, be2, Wm1, bm1, gm1, bem1, Wm2, bm2):
    src = edge_index[0].astype(jnp.int32)
    dst = edge_index[1].astype(jnp.int32)
    pad = _EP - _E
    srcp = jnp.concatenate([src, jnp.zeros((pad,), jnp.int32)])
    dstp = jnp.concatenate([dst, jnp.full((pad,), _N, jnp.int32)])
    srcp = srcp.reshape(_NW, _PB, _BLK)
    dstp = dstp.reshape(_NW, _PB, _BLK)

    ones128 = jnp.ones((_BLK, _H), jnp.float32)
    zeros128 = jnp.zeros((_RPT, _H), jnp.float32)

    degP = _deg_partials(dstp, ones128, zeros128)
    dis, ap0 = _pre_kernel(degP, x, W0)

    P0 = _conv_partials(ap0, srcp, dstp, zeros128)
    ap1, carry1 = _mid_kernel(P0, ap0, b0[None, :], dis, g0, be0, W1, Wr1, b1)

    P1 = _conv_partials(ap1, srcp, dstp, zeros128)
    ap2, carry2 = _mid_kernel(P1, ap1, carry1, dis, g1, be1, W2, Wr2, b2)

    P2 = _conv_partials(ap2, srcp, dstp, zeros128)
    return _post_kernel(P2, ap2, carry2, dis, g2, be2,
                        batch.astype(jnp.int32)[:, None],
                        Wm1, bm1, gm1, bem1, Wm2, bm2)


# speedup vs baseline: 7.5569x; 1.1159x over previous
from functools import partial

import jax
import jax.numpy as jnp
from jax import lax
from jax.experimental import pallas as pl
from jax.experimental.pallas import tpu as pltpu
from jax.experimental.pallas import tpu_sc as plsc

_N = 10000
_E = 320000
_H = 128
_G = 128
_C = 40
_NC = 2
_NS = 16
_NW = _NC * _NS
_BLK = 128
_PB = 80
_CHK = 40
_EPW = _PB * _BLK
_EP = _NW * _EPW
_RPT = 632
_NR = _NS * _RPT
_EPS = 1e-5


def _sc_mesh():
    return plsc.VectorSubcoreMesh(core_axis_name="c", subcore_axis_name="s",
                                  num_cores=_NC, num_subcores=_NS)


def _deg_partials(dstp, ones128, zeros128):
    @partial(
        pl.kernel,
        out_type=jax.ShapeDtypeStruct((2 * _NR, _H), jnp.float32),
        mesh=_sc_mesh(),
        scratch_types=[
            pltpu.VMEM((_PB, _BLK), jnp.int32),
            pltpu.VMEM((_BLK, _H), jnp.float32),
            pltpu.VMEM_SHARED((_NR, _H), jnp.float32),
        ],
    )
    def deg_kernel(dst_hbm, ones_hbm, zer_hbm, out_hbm, dst_v, buf_v, acc_sh):
        c = lax.axis_index("c")
        s = lax.axis_index("s")
        wid = c * _NS + s
        pltpu.sync_copy(zer_hbm, acc_sh.at[pl.ds(s * _RPT, _RPT)])
        pltpu.sync_copy(ones_hbm, buf_v)
        pltpu.sync_copy(dst_hbm.at[wid], dst_v)
        plsc.subcore_barrier()

        def step(j, carry):
            pltpu.sync_copy(buf_v, acc_sh.at[dst_v.at[j]], add=True)
            return carry

        lax.fori_loop(0, _PB, step, 0)
        plsc.subcore_barrier()
        base = c * _NR + s * _RPT
        pltpu.sync_copy(acc_sh.at[pl.ds(s * _RPT, _RPT)],
                        out_hbm.at[pl.ds(base, _RPT)])

    return deg_kernel(dstp, ones128, zeros128)


def _conv_partials(ap, srcp, dstp, zeros128):
    @partial(
        pl.kernel,
        out_type=jax.ShapeDtypeStruct((2 * _NR, _H), jnp.float32),
        mesh=_sc_mesh(),
        scratch_types=[
            pltpu.VMEM((_CHK, _BLK), jnp.int32),
            pltpu.VMEM((_CHK, _BLK), jnp.int32),
            pltpu.VMEM((_BLK, _H), jnp.float32),
            pltpu.VMEM((_BLK, _H), jnp.float32),
            pltpu.VMEM_SHARED((_NR, _H), jnp.float32),
            pltpu.SemaphoreType.DMA,
            pltpu.SemaphoreType.DMA,
        ],
    )
    def conv_kernel(ap_hbm, src_hbm, dst_hbm, zer_hbm, out_hbm,
                    src_v, dst_v, buf0, buf1, acc_sh, sem0, sem1):
        c = lax.axis_index("c")
        s = lax.axis_index("s")
        wid = c * _NS + s
        pltpu.sync_copy(zer_hbm, acc_sh.at[pl.ds(s * _RPT, _RPT)])
        plsc.subcore_barrier()

        for ch in range(_PB // _CHK):
            pltpu.sync_copy(src_hbm.at[wid, pl.ds(ch * _CHK, _CHK)], src_v)
            pltpu.sync_copy(dst_hbm.at[wid, pl.ds(ch * _CHK, _CHK)], dst_v)
            pltpu.async_copy(ap_hbm.at[src_v.at[0]], buf0, sem0)

            def pair(jj, carry):
                j0 = 2 * jj
                j1 = j0 + 1
                pltpu.async_copy(ap_hbm.at[src_v.at[j1]], buf1, sem1)
                pltpu.make_async_copy(ap_hbm.at[src_v.at[j0]], buf0, sem0).wait()
                pltpu.sync_copy(buf0, acc_sh.at[dst_v.at[j0]], add=True)
                pltpu.async_copy(ap_hbm.at[src_v.at[j0 + 2]], buf0, sem0)
                pltpu.make_async_copy(ap_hbm.at[src_v.at[j1]], buf1, sem1).wait()
                pltpu.sync_copy(buf1, acc_sh.at[dst_v.at[j1]], add=True)
                return carry

            lax.fori_loop(0, _CHK // 2 - 1, pair, 0)
            j0 = _CHK - 2
            j1 = _CHK - 1
            pltpu.async_copy(ap_hbm.at[src_v.at[j1]], buf1, sem1)
            pltpu.make_async_copy(ap_hbm.at[src_v.at[j0]], buf0, sem0).wait()
            pltpu.sync_copy(buf0, acc_sh.at[dst_v.at[j0]], add=True)
            pltpu.make_async_copy(ap_hbm.at[src_v.at[j1]], buf1, sem1).wait()
            pltpu.sync_copy(buf1, acc_sh.at[dst_v.at[j1]], add=True)

        plsc.subcore_barrier()
        base = c * _NR + s * _RPT
        pltpu.sync_copy(acc_sh.at[pl.ds(s * _RPT, _RPT)],
                        out_hbm.at[pl.ds(base, _RPT)])

    return conv_kernel(ap, srcp, dstp, zeros128)


def _pre_kernel(deg16, x, W0):
    def body(degr, xr, wr, disr, apr):
        d = degr[...]
        deg = d[0:_N, 0:1] + d[_NR:_NR + _N, 0:1] + 1.0
        dis = lax.rsqrt(deg)
        disr[...] = dis
        a = jnp.dot(xr[...], wr[...], preferred_element_type=jnp.float32)
        apr[...] = a * dis

    return pl.pallas_call(
        body,
        out_shape=[
            jax.ShapeDtypeStruct((_N, 1), jnp.float32),
            jax.ShapeDtypeStruct((_N, _H), jnp.float32),
        ],
    )(deg16, x, W0)


def _mid_kernel(P, ap_prev, carry, dis, g, be, W, Wr, b_next):
    def body(Pr, apr, cr, disr, gr, ber, Wrf, Wrr, bnr, ap_out, carry_out):
        dis = disr[...]
        t = dis * (Pr[0:_N] + Pr[_NR:_NR + _N] + apr[...]) + cr[...]
        m = jnp.mean(t, axis=0)
        v = jnp.mean((t - m) ** 2, axis=0)
        h = jax.nn.relu(gr[...] * (t - m) / jnp.sqrt(v + _EPS) + ber[...])
        ap_out[...] = dis * jnp.dot(h, Wrf[...],
                                    preferred_element_type=jnp.float32)
        carry_out[...] = jnp.dot(h, Wrr[...],
                                 preferred_element_type=jnp.float32) + bnr[...]

    return pl.pallas_call(
        body,
        out_shape=[
            jax.ShapeDtypeStruct((_N, _H), jnp.float32),
            jax.ShapeDtypeStruct((_N, _H), jnp.float32),
        ],
    )(P, ap_prev, carry, dis, g, be, W, Wr, b_next)


def _post_kernel(P, ap2, carry2, dis, g2, be2, batch2d, Wm1, bm1, gm1, bem1, Wm2, bm2):
    def body(Pr, apr, cr, disr, gr, ber, br, Wm1r, bm1r, gm1r, bem1r,
             Wm2r, bm2r, outr, p_ref):
        dis = disr[...]
        t = dis * (Pr[0:_N] + Pr[_NR:_NR + _N] + apr[...]) + cr[...]
        m = jnp.mean(t, axis=0)
        v = jnp.mean((t - m) ** 2, axis=0)
        h = jax.nn.relu(gr[...] * (t - m) / jnp.sqrt(v + _EPS) + ber[...])
        bids = br[...]

        def pool_body(gi, carry):
            mask = bids == gi
            mx = jnp.max(jnp.where(mask, h, -jnp.inf), axis=0)
            p_ref[pl.ds(gi, 1), :] = mx[None, :]
            return carry

        lax.fori_loop(0, _G, pool_body, 0)
        p = p_ref[...]
        m1 = jnp.dot(p, Wm1r[...], preferred_element_type=jnp.float32) + bm1r[...]
        mm = jnp.mean(m1, axis=0)
        vv = jnp.mean((m1 - mm) ** 2, axis=0)
        hm = jax.nn.relu(gm1r[...] * (m1 - mm) / jnp.sqrt(vv + _EPS) + bem1r[...])
        outr[...] = jnp.dot(hm, Wm2r[...],
                            preferred_element_type=jnp.float32) + bm2r[...]

    return pl.pallas_call(
        body,
        out_shape=jax.ShapeDtypeStruct((_G, _C), jnp.float32),
        scratch_shapes=[pltpu.VMEM((_G, _H), jnp.float32)],
    )(P, ap2, carry2, dis, g2, be2, batch2d, Wm1, bm1, gm1, bem1, Wm2, bm2)


def kernel(x, edge_index, batch, W0, b0, g0, be0, W1, b1, Wr1, g1, be1,
           W2, b2, Wr2, g2, be2, Wm1, bm1, gm1, bem1, Wm2, bm2):
    src = edge_index[0].astype(jnp.int32)
    dst = edge_index[1].astype(jnp.int32)
    pad = _EP - _E
    srcp = jnp.concatenate([src, jnp.zeros((pad,), jnp.int32)])
    dstp = jnp.concatenate([dst, jnp.full((pad,), _N, jnp.int32)])
    srcp = srcp.reshape(_NW, _PB, _BLK)
    dstp = dstp.reshape(_NW, _PB, _BLK)

    ones128 = jnp.ones((_BLK, _H), jnp.float32)
    zeros128 = jnp.zeros((_RPT, _H), jnp.float32)

    degP = _deg_partials(dstp, ones128, zeros128)
    dis, ap0 = _pre_kernel(degP, x, W0)

    P0 = _conv_partials(ap0, srcp, dstp, zeros128)
    ap1, carry1 = _mid_kernel(P0, ap0, b0[None, :], dis, g0, be0, W1, Wr1, b1)

    P1 = _conv_partials(ap1, srcp, dstp, zeros128)
    ap2, carry2 = _mid_kernel(P1, ap1, carry1, dis, g1, be1, W2, Wr2, b2)

    P2 = _conv_partials(ap2, srcp, dstp, zeros128)
    return _post_kernel(P2, ap2, carry2, dis, g2, be2,
                        batch.astype(jnp.int32)[:, None],
                        Wm1, bm1, gm1, bem1, Wm2, bm2)
